# baseline (speedup 1.0000x reference)
import functools

import jax
import jax.numpy as jnp
from jax import lax
from jax.experimental import pallas as pl
from jax.experimental.pallas import tpu as pltpu
from jax.experimental.pallas import tpu_sc as plsc

_E = 16
_TOP_K = 2
_TB = 1024
_NW = 32
_TW = 512
_L = 16


def _logits_kernel(x_ref, gw_ref, l3_ref):
    lt = jax.lax.dot_general(
        gw_ref[...], x_ref[...],
        dimension_numbers=(((1,), (1,)), ((), ())),
        preferred_element_type=jnp.float32,
    )
    l3_ref[0] = lt[:, :_TW]
    l3_ref[1] = lt[:, _TW:]


def _zeros_kernel(z_ref):
    z_ref[...] = jnp.zeros_like(z_ref)


def _sc_mask_body(l3_hbm, mask_hbm, lg_v, mk_v):
    wid = lax.axis_index("s") * 2 + lax.axis_index("c")
    pltpu.sync_copy(l3_hbm.at[wid], lg_v)

    def chunk(c, carry):
        t0 = c * _L
        le = [lg_v[e, pl.ds(t0, _L)] for e in range(_E)]
        m1 = le[0]
        i1 = jnp.zeros((_L,), jnp.int32)
        for e in range(1, _E):
            gt = le[e] > m1
            m1 = jnp.where(gt, le[e], m1)
            i1 = jnp.where(gt, e, i1)
        m2 = jnp.full((_L,), -jnp.inf, jnp.float32)
        i2 = jnp.zeros((_L,), jnp.int32)
        for e in range(_E):
            gt = jnp.logical_and(le[e] > m2, i1 != e)
            m2 = jnp.where(gt, le[e], m2)
            i2 = jnp.where(gt, e, i2)
        one = jnp.float32(1.0)
        zero = jnp.float32(0.0)
        for e in range(_E):
            mk_v[e, 0, pl.ds(t0, _L)] = jnp.where(i1 == e, one, zero)
            mk_v[e, 1, pl.ds(t0, _L)] = jnp.where(i2 == e, one, zero)
        return carry

    lax.fori_loop(0, _TW // _L, chunk, 0)
    pltpu.sync_copy(mk_v, mask_hbm.at[:, :, pl.ds(wid * _TW, _TW)])


def kernel(hidden_states, gate_w):
    b, s, h = hidden_states.shape
    t = b * s
    x = hidden_states.reshape(t, h)

    logits3 = pl.pallas_call(
        _logits_kernel,
        grid=(t // _TB,),
        in_specs=[
            pl.BlockSpec((_TB, h), lambda i: (i, 0)),
            pl.BlockSpec((_E, h), lambda i: (0, 0)),
        ],
        out_specs=pl.BlockSpec((2, _E, _TW), lambda i: (i, 0, 0)),
        out_shape=jax.ShapeDtypeStruct((_NW, _E, _TW), jnp.float32),
    )(x, gate_w)

    mask = pl.kernel(
        _sc_mask_body,
        out_type=jax.ShapeDtypeStruct((_E, _TOP_K, t), jnp.float32),
        mesh=plsc.VectorSubcoreMesh(
            core_axis_name="c", subcore_axis_name="s",
            num_cores=2, num_subcores=16,
        ),
        scratch_types=[
            pltpu.VMEM((_E, _TW), jnp.float32),
            pltpu.VMEM((_E, _TOP_K, _TW), jnp.float32),
        ],
    )(logits3)

    zeros = pl.pallas_call(
        _zeros_kernel,
        grid=(t // _TB,),
        out_specs=pl.BlockSpec((_TB, h), lambda i: (i, 0)),
        out_shape=jax.ShapeDtypeStruct((t, h), jnp.float32),
    )()
    return (zeros, mask)


# speedup vs baseline: 1.0037x; 1.0037x over previous
import functools

import jax
import jax.numpy as jnp
from jax import lax
from jax.experimental import pallas as pl
from jax.experimental.pallas import tpu as pltpu
from jax.experimental.pallas import tpu_sc as plsc

_E = 16
_TOP_K = 2
_TB = 1024
_NW = 32
_TW = 512
_L = 16


def _logits_kernel(x_ref, gw_ref, l3_ref):
    lt = jax.lax.dot_general(
        gw_ref[...], x_ref[...],
        dimension_numbers=(((1,), (1,)), ((), ())),
        preferred_element_type=jnp.float32,
    )
    l3_ref[0] = lt[:, :_TW]
    l3_ref[1] = lt[:, _TW:]


def _zeros_kernel(z_ref):
    z_ref[...] = jnp.zeros_like(z_ref)


def _sc_mask_body(l3_hbm, mask_hbm, lg_v, mk_v):
    wid = lax.axis_index("s") * 2 + lax.axis_index("c")
    pltpu.sync_copy(l3_hbm.at[wid], lg_v)

    def chunk(c, carry):
        t0 = c * _L
        le = [lg_v[e, pl.ds(t0, _L)] for e in range(_E)]
        m1 = le[0]
        i1 = jnp.zeros((_L,), jnp.int32)
        for e in range(1, _E):
            gt = le[e] > m1
            m1 = jnp.where(gt, le[e], m1)
            i1 = jnp.where(gt, e, i1)
        m2 = jnp.full((_L,), -jnp.inf, jnp.float32)
        i2 = jnp.zeros((_L,), jnp.int32)
        for e in range(_E):
            gt = jnp.logical_and(le[e] > m2, i1 != e)
            m2 = jnp.where(gt, le[e], m2)
            i2 = jnp.where(gt, e, i2)
        one = jnp.float32(1.0)
        zero = jnp.float32(0.0)
        for e in range(_E):
            mk_v[e, 0, pl.ds(t0, _L)] = jnp.where(i1 == e, one, zero)
            mk_v[e, 1, pl.ds(t0, _L)] = jnp.where(i2 == e, one, zero)
        return carry

    lax.fori_loop(0, _TW // _L, chunk, 0)
    pltpu.sync_copy(mk_v, mask_hbm.at[:, :, pl.ds(wid * _TW, _TW)])


def kernel(hidden_states, gate_w):
    b, s, h = hidden_states.shape
    t = b * s
    x = hidden_states.reshape(t, h)

    logits3 = pl.pallas_call(
        _logits_kernel,
        grid=(t // _TB,),
        in_specs=[
            pl.BlockSpec((_TB, h), lambda i: (i, 0)),
            pl.BlockSpec((_E, h), lambda i: (0, 0)),
        ],
        out_specs=pl.BlockSpec((2, _E, _TW), lambda i: (i, 0, 0)),
        out_shape=jax.ShapeDtypeStruct((_NW, _E, _TW), jnp.float32),
    )(x, gate_w)

    zeros = pl.pallas_call(
        _zeros_kernel,
        grid=(t // _TB,),
        out_specs=pl.BlockSpec((_TB, h), lambda i: (i, 0)),
        out_shape=jax.ShapeDtypeStruct((t, h), jnp.float32),
    )()

    mask = pl.kernel(
        _sc_mask_body,
        out_type=jax.ShapeDtypeStruct((_E, _TOP_K, t), jnp.float32),
        mesh=plsc.VectorSubcoreMesh(
            core_axis_name="c", subcore_axis_name="s",
            num_cores=2, num_subcores=16,
        ),
        scratch_types=[
            pltpu.VMEM((_E, _TW), jnp.float32),
            pltpu.VMEM((_E, _TOP_K, _TW), jnp.float32),
        ],
    )(logits3)
    return (zeros, mask)


# speedup vs baseline: 1.1824x; 1.1780x over previous
import jax
import jax.numpy as jnp
from jax.experimental import pallas as pl
from jax.experimental.pallas import tpu as pltpu

_TOP_K = 2
_TB = 2048


def _routing_kernel(x_ref, gw_ref, z_ref, m_ref, zb_ref, sem):
    i = pl.program_id(0)
    n = pl.num_programs(0)

    @pl.when(i == 0)
    def _init():
        zb_ref[...] = jnp.zeros_like(zb_ref)

    @pl.when(i > 0)
    def _drain_prev():
        pltpu.make_async_copy(
            zb_ref, z_ref.at[pl.ds((i - 1) * _TB, _TB), :], sem
        ).wait()

    cur = pltpu.make_async_copy(
        zb_ref, z_ref.at[pl.ds(i * _TB, _TB), :], sem
    )
    cur.start()

    lt = jax.lax.dot_general(
        gw_ref[...], x_ref[...],
        dimension_numbers=(((1,), (1,)), ((), ())),
        preferred_element_type=jnp.float32,
    )
    E = lt.shape[0]
    eidx = jax.lax.broadcasted_iota(jnp.int32, lt.shape, 0)
    m1 = jnp.max(lt, axis=0, keepdims=True)
    i1 = jnp.min(jnp.where(lt == m1, eidx, E), axis=0, keepdims=True)
    lt2 = jnp.where(eidx == i1, -jnp.inf, lt)
    m2 = jnp.max(lt2, axis=0, keepdims=True)
    i2 = jnp.min(jnp.where(lt2 == m2, eidx, E), axis=0, keepdims=True)
    e3 = jax.lax.broadcasted_iota(jnp.int32, m_ref.shape, 0)
    k3 = jax.lax.broadcasted_iota(jnp.int32, m_ref.shape, 1)
    sel = jnp.where(k3 == 0, i1[None], i2[None])
    m_ref[...] = (e3 == sel).astype(jnp.float32)

    @pl.when(i == n - 1)
    def _drain_last():
        cur.wait()


def kernel(hidden_states, gate_w):
    b, s, h = hidden_states.shape
    t = b * s
    e = gate_w.shape[0]
    x = hidden_states.reshape(t, h)
    grid = (t // _TB,)
    z, m = pl.pallas_call(
        _routing_kernel,
        grid=grid,
        in_specs=[
            pl.BlockSpec((_TB, h), lambda i: (i, 0)),
            pl.BlockSpec((e, h), lambda i: (0, 0)),
        ],
        out_specs=[
            pl.BlockSpec(memory_space=pl.ANY),
            pl.BlockSpec((e, _TOP_K, _TB), lambda i: (0, 0, i)),
        ],
        out_shape=[
            jax.ShapeDtypeStruct((t, h), jnp.float32),
            jax.ShapeDtypeStruct((e, _TOP_K, t), jnp.float32),
        ],
        scratch_shapes=[
            pltpu.VMEM((_TB, h), jnp.float32),
            pltpu.SemaphoreType.DMA,
        ],
    )(x, gate_w)
    return (z, m)


# speedup vs baseline: 1.2034x; 1.0177x over previous
import jax
import jax.numpy as jnp
from jax.experimental import pallas as pl
from jax.experimental.pallas import tpu as pltpu

_TOP_K = 2
_TB = 1024


def _routing_kernel(x_ref, gw_ref, z_ref, m_ref):
    z_ref[...] = jnp.zeros_like(z_ref)
    lt = jax.lax.dot_general(
        gw_ref[...], x_ref[...],
        dimension_numbers=(((1,), (1,)), ((), ())),
        preferred_element_type=jnp.float32,
    )
    E = lt.shape[0]
    eidx = jax.lax.broadcasted_iota(jnp.int32, lt.shape, 0)
    m1 = jnp.max(lt, axis=0, keepdims=True)
    i1 = jnp.min(jnp.where(lt == m1, eidx, E), axis=0, keepdims=True)
    lt2 = jnp.where(eidx == i1, -jnp.inf, lt)
    m2 = jnp.max(lt2, axis=0, keepdims=True)
    i2 = jnp.min(jnp.where(lt2 == m2, eidx, E), axis=0, keepdims=True)
    e3 = jax.lax.broadcasted_iota(jnp.int32, m_ref.shape, 0)
    k3 = jax.lax.broadcasted_iota(jnp.int32, m_ref.shape, 1)
    sel = jnp.where(k3 == 0, i1[None], i2[None])
    m_ref[...] = (e3 == sel).astype(jnp.float32)


def kernel(hidden_states, gate_w):
    b, s, h = hidden_states.shape
    t = b * s
    e = gate_w.shape[0]
    x = hidden_states.reshape(t, h)
    grid = (t // _TB,)
    z, m = pl.pallas_call(
        _routing_kernel,
        grid=grid,
        in_specs=[
            pl.BlockSpec((_TB, h), lambda i: (i, 0)),
            pl.BlockSpec((e, h), lambda i: (0, 0)),
        ],
        out_specs=[
            pl.BlockSpec((_TB, h), lambda i: (i, 0)),
            pl.BlockSpec((e, _TOP_K, _TB), lambda i: (0, 0, i)),
        ],
        out_shape=[
            jax.ShapeDtypeStruct((t, h), jnp.float32),
            jax.ShapeDtypeStruct((e, _TOP_K, t), jnp.float32),
        ],
        compiler_params=pltpu.CompilerParams(
            dimension_semantics=("parallel",),
        ),
    )(x, gate_w)
    return (z, m)
